# Initial kernel scaffold; baseline (speedup 1.0000x reference)
import functools

import jax
import jax.numpy as jnp
from jax import lax
from jax.experimental import pallas as pl
from jax.experimental.pallas import tpu as pltpu
from jax.experimental.pallas import tpu_sc as plsc

_N = 10000
_E = 160000
_H = 256
_R = 16
_BD = _H - _R
_NC = 2
_NS = 16
_LANES = 16
_HALF = _H // _NC
_VPR = _HALF // _LANES
_EPT = _E // _NS
_KC = 80
_NCHUNK = _EPT // _KC
_RPT = _N // _NS
_RCH = 125
_NRCH = _RPT // _RCH


def _sc_message_passing(x3, x2r, edge_index, ea_t, rand_edge, embc_r, eps16):
  mesh = plsc.VectorSubcoreMesh(core_axis_name="c", subcore_axis_name="s")

  @functools.partial(
      pl.kernel,
      out_type=jax.ShapeDtypeStruct((_N, _NC, _HALF), jnp.float32),
      mesh=mesh,
      scratch_types=[
          pltpu.VMEM_SHARED((_N, _HALF), jnp.float32),
          pltpu.VMEM((_RCH, _HALF), jnp.float32),
          pltpu.VMEM((_KC, _HALF), jnp.float32),
          pltpu.VMEM((_KC, _R), jnp.float32),
          pltpu.VMEM((_KC,), jnp.int32),
          pltpu.VMEM((_KC,), jnp.int32),
          pltpu.VMEM((_KC,), jnp.int32),
          pltpu.VMEM((_KC,), jnp.int32),
          pltpu.VMEM((_KC,), jnp.int32),
          pltpu.VMEM((_KC,), jnp.int32),
          pltpu.VMEM((_KC,), jnp.int32),
          pltpu.VMEM((_LANES,), jnp.float32),
          pltpu.SemaphoreType.DMA,
          pltpu.SemaphoreType.DMA,
      ],
  )
  def k(x3h, x2rh, eih, eath, randh, embch, epsh, outh,
        acc, ibuf, mbuf, rbuf, srcb, dstb, a0b, a1b, a2b, gsrc, gemb, epsv,
        sem1, sem2):
    c = lax.axis_index("c")
    s = lax.axis_index("s")
    cf = c.astype(jnp.float32)
    pltpu.sync_copy(epsh, epsv)
    e1 = epsv[:] + 1.0

    r0 = s * _RPT

    def init_chunk(kk, _):
      rr = r0 + kk * _RCH
      pltpu.sync_copy(x3h.at[pl.ds(rr, _RCH), c], ibuf)

      def irow(i, _):
        for v in range(_VPR):
          sl = pl.ds(v * _LANES, _LANES)
          ibuf[i, sl] = ibuf[i, sl] * e1
        return 0

      lax.fori_loop(0, _RCH, irow, 0)
      pltpu.sync_copy(ibuf, acc.at[pl.ds(rr, _RCH)])
      return 0

    lax.fori_loop(0, _NRCH, init_chunk, 0)
    plsc.subcore_barrier()

    e0 = s * _EPT

    def chunk(j, _):
      off = e0 + j * _KC
      pltpu.sync_copy(eih.at[0, pl.ds(off, _KC)], srcb)
      pltpu.sync_copy(eih.at[1, pl.ds(off, _KC)], dstb)
      pltpu.sync_copy(eath.at[0, pl.ds(off, _KC)], a0b)
      pltpu.sync_copy(eath.at[1, pl.ds(off, _KC)], a1b)
      pltpu.sync_copy(eath.at[2, pl.ds(off, _KC)], a2b)
      pltpu.sync_copy(randh.at[pl.ds(off, _KC)], rbuf)
      for i in range(_KC // _LANES):
        sl = pl.ds(i * _LANES, _LANES)
        gsrc[sl] = srcb[sl] * 2 + c
        gemb[sl] = (a0b[sl] * 12 + a1b[sl] * 2 + a2b[sl]) * 2 + c
      pltpu.async_copy(x2rh.at[gsrc], mbuf, sem1).wait()
      pltpu.async_copy(embch.at[gemb], mbuf, sem2, add=True).wait()

      def rrow(r, _):
        for v in range(_VPR):
          sl = pl.ds(v * _LANES, _LANES)
          val = mbuf[r, sl]
          if v == _VPR - 1:
            val = val + rbuf[r, :] * cf
          mbuf[r, sl] = jnp.maximum(val, 0.0)
        return 0

      lax.fori_loop(0, _KC, rrow, 0)
      pltpu.sync_copy(mbuf, acc.at[dstb], add=True)
      return 0

    lax.fori_loop(0, _NCHUNK, chunk, 0)
    plsc.subcore_barrier()

    pltpu.sync_copy(acc.at[pl.ds(r0, _RPT)], outh.at[pl.ds(r0, _RPT), c])

  return k(x3, x2r, edge_index, ea_t, rand_edge, embc_r, eps16)


_NB = 10
_BR = _N // _NB


def _mlp1(h, W1, b1):
  def body(h_ref, w_ref, b_ref, h1_ref, sums_ref, accs):
    i = pl.program_id(0)
    h1 = jnp.dot(h_ref[:], w_ref[:], preferred_element_type=jnp.float32)
    h1 = h1 + b_ref[:]
    h1_ref[:] = h1

    @pl.when(i == 0)
    def _():
      accs[:] = jnp.zeros_like(accs)

    accs[0:1, :] = accs[0:1, :] + jnp.sum(h1, axis=0, keepdims=True)
    accs[1:2, :] = accs[1:2, :] + jnp.sum(h1 * h1, axis=0, keepdims=True)
    sums_ref[:] = accs[:]

  return pl.pallas_call(
      body,
      grid=(_NB,),
      in_specs=[
          pl.BlockSpec((_BR, _H), lambda i: (i, 0)),
          pl.BlockSpec((_H, _H), lambda i: (0, 0)),
          pl.BlockSpec((1, _H), lambda i: (0, 0)),
      ],
      out_specs=[
          pl.BlockSpec((_BR, _H), lambda i: (i, 0)),
          pl.BlockSpec((8, _H), lambda i: (0, 0)),
      ],
      out_shape=[
          jax.ShapeDtypeStruct((_N, _H), jnp.float32),
          jax.ShapeDtypeStruct((8, _H), jnp.float32),
      ],
      scratch_shapes=[pltpu.VMEM((8, _H), jnp.float32)],
  )(h, W1, b1.reshape(1, _H))


def _mlp2(h1, sums, gamma, beta, W2, b2):
  def body(h1_ref, sums_ref, g_ref, be_ref, w_ref, b_ref, o_ref):
    mu = sums_ref[0:1, :] / _N
    ms = sums_ref[1:2, :] / _N
    var = ms - mu * mu
    inv = lax.rsqrt(var + 1e-5)
    a = (h1_ref[:] - mu) * (inv * g_ref[:]) + be_ref[:]
    a = jnp.maximum(a, 0.0)
    o_ref[:] = jnp.dot(a, w_ref[:], preferred_element_type=jnp.float32) + b_ref[:]

  return pl.pallas_call(
      body,
      grid=(_NB,),
      in_specs=[
          pl.BlockSpec((_BR, _H), lambda i: (i, 0)),
          pl.BlockSpec((8, _H), lambda i: (0, 0)),
          pl.BlockSpec((1, _H), lambda i: (0, 0)),
          pl.BlockSpec((1, _H), lambda i: (0, 0)),
          pl.BlockSpec((_H, _H), lambda i: (0, 0)),
          pl.BlockSpec((1, _H), lambda i: (0, 0)),
      ],
      out_specs=pl.BlockSpec((_BR, _H), lambda i: (i, 0)),
      out_shape=jax.ShapeDtypeStruct((_N, _H), jnp.float32),
  )(h1, sums, gamma.reshape(1, _H), beta.reshape(1, _H), W2, b2.reshape(1, _H))


def kernel(x, edge_index, edge_attr, rand_edge, emb0, emb1, emb2,
           W1, b1, gamma, beta, W2, b2, eps):
  x3 = x.reshape(_N, _NC, _HALF)
  x2r = x.reshape(_N * _NC, _HALF)
  ea_t = edge_attr.T
  embc = (emb0[:, None, None, :] + emb1[None, :, None, :]
          + emb2[None, None, :, :]).reshape(5 * 6 * 2, _BD)
  embc = jnp.concatenate([embc, jnp.zeros((5 * 6 * 2, _R), jnp.float32)],
                         axis=1)
  embc_r = embc.reshape(2 * 5 * 6 * 2, _HALF)
  eps16 = jnp.broadcast_to(eps, (_LANES,)).astype(jnp.float32)

  h = _sc_message_passing(x3, x2r, edge_index, ea_t, rand_edge, embc_r, eps16)
  h = h.reshape(_N, _H)
  h1, sums = _mlp1(h, W1, b1)
  return _mlp2(h1, sums, gamma, beta, W2, b2)


# speedup vs baseline: 2.2604x; 2.2604x over previous
import functools

import jax
import jax.numpy as jnp
from jax import lax
from jax.experimental import pallas as pl
from jax.experimental.pallas import tpu as pltpu
from jax.experimental.pallas import tpu_sc as plsc

_N = 10000
_E = 160000
_H = 256
_R = 16
_BD = _H - _R
_NC = 2
_NS = 16
_LANES = 16
_HALF = _H // _NC
_VPR = _HALF // _LANES
_EPT = _E // _NS
_KC = 80
_NCHUNK = _EPT // _KC
_RPT = _N // _NS
_RCH = 125
_NRCH = _RPT // _RCH


def _sc_message_passing(x3, x2r, src, dst, a0, a1, a2, rand_edge, embc_r,
                        eps16):
  mesh = plsc.VectorSubcoreMesh(core_axis_name="c", subcore_axis_name="s")

  @functools.partial(
      pl.kernel,
      out_type=jax.ShapeDtypeStruct((_N, _NC, _HALF), jnp.float32),
      mesh=mesh,
      scratch_types=[
          pltpu.VMEM_SHARED((_N, _HALF), jnp.float32),
          pltpu.VMEM((_RCH, _HALF), jnp.float32),
          pltpu.VMEM((_KC, _HALF), jnp.float32),
          pltpu.VMEM((_KC, _R), jnp.float32),
          pltpu.VMEM((_KC,), jnp.int32),
          pltpu.VMEM((_KC,), jnp.int32),
          pltpu.VMEM((_KC,), jnp.int32),
          pltpu.VMEM((_KC,), jnp.int32),
          pltpu.VMEM((_KC,), jnp.int32),
          pltpu.VMEM((_KC,), jnp.int32),
          pltpu.VMEM((_KC,), jnp.int32),
          pltpu.VMEM((_LANES,), jnp.float32),
          pltpu.SemaphoreType.DMA,
          pltpu.SemaphoreType.DMA,
      ],
  )
  def k(x3h, x2rh, srch, dsth, a0h, a1h, a2h, randh, embch, epsh, outh,
        acc, ibuf, mbuf, rbuf, srcb, dstb, a0b, a1b, a2b, gsrc, gemb, epsv,
        sem1, sem2):
    c = lax.axis_index("c")
    s = lax.axis_index("s")
    cf = c.astype(jnp.float32)
    pltpu.sync_copy(epsh, epsv)
    e1 = epsv[:] + 1.0

    r0 = s * _RPT

    def init_chunk(kk, _):
      rr = r0 + kk * _RCH
      pltpu.sync_copy(x3h.at[pl.ds(rr, _RCH), c], ibuf)

      def irow(i, _):
        for v in range(_VPR):
          sl = pl.ds(v * _LANES, _LANES)
          ibuf[i, sl] = ibuf[i, sl] * e1
        return 0

      lax.fori_loop(0, _RCH, irow, 0)
      pltpu.sync_copy(ibuf, acc.at[pl.ds(rr, _RCH)])
      return 0

    lax.fori_loop(0, _NRCH, init_chunk, 0)
    plsc.subcore_barrier()

    e0 = s * _EPT

    def chunk(j, _):
      off = e0 + j * _KC
      pltpu.sync_copy(srch.at[pl.ds(off, _KC)], srcb)
      pltpu.sync_copy(dsth.at[pl.ds(off, _KC)], dstb)
      pltpu.sync_copy(a0h.at[pl.ds(off, _KC)], a0b)
      pltpu.sync_copy(a1h.at[pl.ds(off, _KC)], a1b)
      pltpu.sync_copy(a2h.at[pl.ds(off, _KC)], a2b)
      pltpu.sync_copy(randh.at[pl.ds(off, _KC)], rbuf)
      for i in range(_KC // _LANES):
        sl = pl.ds(i * _LANES, _LANES)
        gsrc[sl] = srcb[sl] * 2 + c
        gemb[sl] = (a0b[sl] * 12 + a1b[sl] * 2 + a2b[sl]) * 2 + c
      pltpu.async_copy(x2rh.at[gsrc], mbuf, sem1).wait()
      pltpu.async_copy(embch.at[gemb], mbuf, sem2, add=True).wait()

      def rrow(r, _):
        for v in range(_VPR):
          sl = pl.ds(v * _LANES, _LANES)
          val = mbuf[r, sl]
          if v == _VPR - 1:
            val = val + rbuf[r, :] * cf
          mbuf[r, sl] = jnp.maximum(val, 0.0)
        return 0

      lax.fori_loop(0, _KC, rrow, 0)
      pltpu.sync_copy(mbuf, acc.at[dstb], add=True)
      return 0

    lax.fori_loop(0, _NCHUNK, chunk, 0)
    plsc.subcore_barrier()

    pltpu.sync_copy(acc.at[pl.ds(r0, _RPT)], outh.at[pl.ds(r0, _RPT), c])

  return k(x3, x2r, src, dst, a0, a1, a2, rand_edge, embc_r, eps16)


_NB = 10
_BR = _N // _NB


def _mlp1(h, W1, b1):
  def body(h_ref, w_ref, b_ref, h1_ref, sums_ref, accs):
    i = pl.program_id(0)
    h1 = jnp.dot(h_ref[:], w_ref[:], preferred_element_type=jnp.float32)
    h1 = h1 + b_ref[:]
    h1_ref[:] = h1

    @pl.when(i == 0)
    def _():
      accs[:] = jnp.zeros_like(accs)

    accs[0:1, :] = accs[0:1, :] + jnp.sum(h1, axis=0, keepdims=True)
    accs[1:2, :] = accs[1:2, :] + jnp.sum(h1 * h1, axis=0, keepdims=True)
    sums_ref[:] = accs[:]

  return pl.pallas_call(
      body,
      grid=(_NB,),
      in_specs=[
          pl.BlockSpec((_BR, _H), lambda i: (i, 0)),
          pl.BlockSpec((_H, _H), lambda i: (0, 0)),
          pl.BlockSpec((1, _H), lambda i: (0, 0)),
      ],
      out_specs=[
          pl.BlockSpec((_BR, _H), lambda i: (i, 0)),
          pl.BlockSpec((8, _H), lambda i: (0, 0)),
      ],
      out_shape=[
          jax.ShapeDtypeStruct((_N, _H), jnp.float32),
          jax.ShapeDtypeStruct((8, _H), jnp.float32),
      ],
      scratch_shapes=[pltpu.VMEM((8, _H), jnp.float32)],
  )(h, W1, b1.reshape(1, _H))


def _mlp2(h1, sums, gamma, beta, W2, b2):
  def body(h1_ref, sums_ref, g_ref, be_ref, w_ref, b_ref, o_ref):
    mu = sums_ref[0:1, :] / _N
    ms = sums_ref[1:2, :] / _N
    var = ms - mu * mu
    inv = lax.rsqrt(var + 1e-5)
    a = (h1_ref[:] - mu) * (inv * g_ref[:]) + be_ref[:]
    a = jnp.maximum(a, 0.0)
    o_ref[:] = jnp.dot(a, w_ref[:], preferred_element_type=jnp.float32) + b_ref[:]

  return pl.pallas_call(
      body,
      grid=(_NB,),
      in_specs=[
          pl.BlockSpec((_BR, _H), lambda i: (i, 0)),
          pl.BlockSpec((8, _H), lambda i: (0, 0)),
          pl.BlockSpec((1, _H), lambda i: (0, 0)),
          pl.BlockSpec((1, _H), lambda i: (0, 0)),
          pl.BlockSpec((_H, _H), lambda i: (0, 0)),
          pl.BlockSpec((1, _H), lambda i: (0, 0)),
      ],
      out_specs=pl.BlockSpec((_BR, _H), lambda i: (i, 0)),
      out_shape=jax.ShapeDtypeStruct((_N, _H), jnp.float32),
  )(h1, sums, gamma.reshape(1, _H), beta.reshape(1, _H), W2, b2.reshape(1, _H))


def kernel(x, edge_index, edge_attr, rand_edge, emb0, emb1, emb2,
           W1, b1, gamma, beta, W2, b2, eps):
  x3 = x.reshape(_N, _NC, _HALF)
  x2r = x.reshape(_N * _NC, _HALF)
  src = edge_index[0]
  dst = edge_index[1]
  a0 = edge_attr[:, 0]
  a1 = edge_attr[:, 1]
  a2 = edge_attr[:, 2]
  embc = (emb0[:, None, None, :] + emb1[None, :, None, :]
          + emb2[None, None, :, :]).reshape(5 * 6 * 2, _BD)
  embc = jnp.concatenate([embc, jnp.zeros((5 * 6 * 2, _R), jnp.float32)],
                         axis=1)
  embc_r = embc.reshape(2 * 5 * 6 * 2, _HALF)
  eps16 = jnp.broadcast_to(eps, (_LANES,)).astype(jnp.float32)

  h = _sc_message_passing(x3, x2r, src, dst, a0, a1, a2, rand_edge, embc_r,
                          eps16)
  h = h.reshape(_N, _H)
  h1, sums = _mlp1(h, W1, b1)
  return _mlp2(h1, sums, gamma, beta, W2, b2)


# speedup vs baseline: 2.5061x; 1.1087x over previous
import functools

import jax
import jax.numpy as jnp
from jax import lax
from jax.experimental import pallas as pl
from jax.experimental.pallas import tpu as pltpu
from jax.experimental.pallas import tpu_sc as plsc

_N = 10000
_E = 160000
_H = 256
_R = 16
_BD = _H - _R
_NC = 2
_NS = 16
_LANES = 16
_HALF = _H // _NC
_VPR = _HALF // _LANES
_EPT = _E // _NS
_KC = 80
_NCHUNK = _EPT // _KC
_RPT = _N // _NS


def _sc_message_passing(x3, x2r, src3, dst3, a03, a13, a23, rand4, embc_r,
                        eps16):
  mesh = plsc.VectorSubcoreMesh(core_axis_name="c", subcore_axis_name="s")

  @functools.partial(
      pl.kernel,
      out_type=jax.ShapeDtypeStruct((_N, _NC, _HALF), jnp.float32),
      mesh=mesh,
      scratch_types=[
          pltpu.VMEM_SHARED((_N, _HALF), jnp.float32),
          pltpu.VMEM((5, _KC), jnp.int32),
          pltpu.VMEM((5, _KC), jnp.int32),
          pltpu.VMEM((2, _KC), jnp.int32),
          pltpu.VMEM((2, _KC), jnp.int32),
          pltpu.VMEM((_KC,), jnp.int32),
          pltpu.VMEM((_KC,), jnp.int32),
          pltpu.VMEM((_KC, _HALF), jnp.float32),
          pltpu.VMEM((_KC, _HALF), jnp.float32),
          pltpu.VMEM((_KC, _R), jnp.float32),
          pltpu.VMEM((_KC, _R), jnp.float32),
          pltpu.VMEM((_LANES,), jnp.float32),
          pltpu.SemaphoreType.DMA,
          pltpu.SemaphoreType.DMA,
          pltpu.SemaphoreType.DMA,
          pltpu.SemaphoreType.DMA,
          pltpu.SemaphoreType.DMA,
          pltpu.SemaphoreType.DMA,
          pltpu.SemaphoreType.DMA,
          pltpu.SemaphoreType.DMA,
      ],
  )
  def k(x3h, x2rh, src3h, dst3h, a03h, a13h, a23h, randh, embch, epsh, outh,
        acc, idx0, idx1, gidx0, gidx1, dstb0, dstb1,
        mbuf0, mbuf1, rbuf0, rbuf1, epsv,
        si0, si1, sx0, sx1, se0, se1, sr0, sr1):
    c = lax.axis_index("c")
    s = lax.axis_index("s")
    cf = c.astype(jnp.float32)
    pltpu.sync_copy(epsh, epsv)
    e1 = epsv[:] + 1.0

    r0 = s * _RPT

    def init_block(rr, nrows):
      pltpu.sync_copy(x3h.at[pl.ds(rr, nrows), c], mbuf0.at[pl.ds(0, nrows)])

      def irow(i, _):
        for v in range(_VPR):
          sl = pl.ds(v * _LANES, _LANES)
          mbuf0[i, sl] = mbuf0[i, sl] * e1
        return 0

      lax.fori_loop(0, nrows, irow, 0)
      pltpu.sync_copy(mbuf0.at[pl.ds(0, nrows)], acc.at[pl.ds(rr, nrows)])

    for kk in range(_RPT // _KC):
      init_block(r0 + kk * _KC, _KC)
    init_block(r0 + (_RPT // _KC) * _KC, _RPT % _KC)
    plsc.subcore_barrier()

    e0 = s * _EPT

    def issue_idx(j, ib, si):
      off = e0 + jnp.minimum(j, _NCHUNK - 1) * _KC
      pltpu.async_copy(src3h.at[pl.ds(off, _KC)], ib.at[0], si)
      pltpu.async_copy(dst3h.at[pl.ds(off, _KC)], ib.at[1], si)
      pltpu.async_copy(a03h.at[pl.ds(off, _KC)], ib.at[2], si)
      pltpu.async_copy(a13h.at[pl.ds(off, _KC)], ib.at[3], si)
      pltpu.async_copy(a23h.at[pl.ds(off, _KC)], ib.at[4], si)

    def wait_idx(ib, si):
      for r in range(5):
        pltpu.make_async_copy(src3h.at[pl.ds(0, _KC)], ib.at[r], si).wait()

    def compute_gidx(ib, gb, db):
      for i in range(_KC // _LANES):
        sl = pl.ds(i * _LANES, _LANES)
        gb[0, sl] = ib[0, sl] * 2 + c
        gb[1, sl] = ((ib[2, sl] * 6 + ib[3, sl]) * 2 + ib[4, sl]) * 2 + c
        db[sl] = ib[1, sl]

    def issue_xr(j, gb, mb, rb, sx, sr):
      pltpu.async_copy(x2rh.at[gb.at[0]], mb, sx)
      pltpu.async_copy(randh.at[pl.ds(e0 + j * _KC, _KC)], rb, sr)

    def wait_x(gb, mb, sx):
      pltpu.make_async_copy(x2rh.at[gb.at[0]], mb, sx).wait()

    def issue_emb(gb, mb, se):
      pltpu.async_copy(embch.at[gb.at[1]], mb, se, add=True)

    def wait_emb(gb, mb, se):
      pltpu.make_async_copy(embch.at[gb.at[1]], mb, se).wait()

    def wait_rand(rb, sr):
      pltpu.make_async_copy(randh.at[pl.ds(0, _KC)], rb, sr).wait()

    def relu_scatter(mb, rb, db):
      def rrow(r, _):
        for v in range(_VPR):
          sl = pl.ds(v * _LANES, _LANES)
          val = mb[r, sl]
          if v == _VPR - 1:
            val = val + rb[r, :] * cf
          mb[r, sl] = jnp.maximum(val, 0.0)
        return 0

      lax.fori_loop(0, _KC, rrow, 0)
      pltpu.sync_copy(mb, acc.at[db], add=True)

    P0 = (idx0, gidx0, dstb0, mbuf0, rbuf0, si0, sx0, se0, sr0)
    P1 = (idx1, gidx1, dstb1, mbuf1, rbuf1, si1, sx1, se1, sr1)

    def half(j, B, Bo):
      (ib, gb, db, mb, rb, si, sx, se, sr) = B
      (ibo, gbo, dbo, mbo, rbo, sio, sxo, seo, sro) = Bo
      wait_idx(ibo, sio)
      compute_gidx(ibo, gbo, dbo)
      issue_xr(j + 1, gbo, mbo, rbo, sxo, sro)
      issue_idx(j + 2, ib, si)
      wait_emb(gb, mb, se)
      wait_rand(rb, sr)
      relu_scatter(mb, rb, db)
      wait_x(gbo, mbo, sxo)
      issue_emb(gbo, mbo, seo)

    pltpu.sync_copy(src3h.at[pl.ds(e0, _KC)], idx0.at[0])
    pltpu.sync_copy(dst3h.at[pl.ds(e0, _KC)], idx0.at[1])
    pltpu.sync_copy(a03h.at[pl.ds(e0, _KC)], idx0.at[2])
    pltpu.sync_copy(a13h.at[pl.ds(e0, _KC)], idx0.at[3])
    pltpu.sync_copy(a23h.at[pl.ds(e0, _KC)], idx0.at[4])
    compute_gidx(idx0, gidx0, dstb0)
    issue_xr(0, gidx0, mbuf0, rbuf0, sx0, sr0)
    issue_idx(1, idx1, si1)
    wait_x(gidx0, mbuf0, sx0)
    issue_emb(gidx0, mbuf0, se0)

    def pair(g, _):
      j0 = g * 2
      half(j0, P0, P1)
      half(j0 + 1, P1, P0)
      return 0

    lax.fori_loop(0, (_NCHUNK - 1) // 2, pair, 0)
    wait_emb(gidx0, mbuf0, se0)
    wait_rand(rbuf0, sr0)
    relu_scatter(mbuf0, rbuf0, dstb0)
    wait_idx(idx1, si1)

    plsc.subcore_barrier()

    pltpu.sync_copy(acc.at[pl.ds(r0, _RPT)], outh.at[pl.ds(r0, _RPT), c])

  return k(x3, x2r, src3, dst3, a03, a13, a23, rand4, embc_r, eps16)


_NB = 10
_BR = _N // _NB


def _mlp1(h, W1, b1):
  def body(h_ref, w_ref, b_ref, h1_ref, sums_ref, accs):
    i = pl.program_id(0)
    h1 = jnp.dot(h_ref[:], w_ref[:], preferred_element_type=jnp.float32)
    h1 = h1 + b_ref[:]
    h1_ref[:] = h1

    @pl.when(i == 0)
    def _():
      accs[:] = jnp.zeros_like(accs)

    accs[0:1, :] = accs[0:1, :] + jnp.sum(h1, axis=0, keepdims=True)
    accs[1:2, :] = accs[1:2, :] + jnp.sum(h1 * h1, axis=0, keepdims=True)
    sums_ref[:] = accs[:]

  return pl.pallas_call(
      body,
      grid=(_NB,),
      in_specs=[
          pl.BlockSpec((_BR, _H), lambda i: (i, 0)),
          pl.BlockSpec((_H, _H), lambda i: (0, 0)),
          pl.BlockSpec((1, _H), lambda i: (0, 0)),
      ],
      out_specs=[
          pl.BlockSpec((_BR, _H), lambda i: (i, 0)),
          pl.BlockSpec((8, _H), lambda i: (0, 0)),
      ],
      out_shape=[
          jax.ShapeDtypeStruct((_N, _H), jnp.float32),
          jax.ShapeDtypeStruct((8, _H), jnp.float32),
      ],
      scratch_shapes=[pltpu.VMEM((8, _H), jnp.float32)],
  )(h, W1, b1.reshape(1, _H))


def _mlp2(h1, sums, gamma, beta, W2, b2):
  def body(h1_ref, sums_ref, g_ref, be_ref, w_ref, b_ref, o_ref):
    mu = sums_ref[0:1, :] / _N
    ms = sums_ref[1:2, :] / _N
    var = ms - mu * mu
    inv = lax.rsqrt(var + 1e-5)
    a = (h1_ref[:] - mu) * (inv * g_ref[:]) + be_ref[:]
    a = jnp.maximum(a, 0.0)
    o_ref[:] = jnp.dot(a, w_ref[:], preferred_element_type=jnp.float32) + b_ref[:]

  return pl.pallas_call(
      body,
      grid=(_NB,),
      in_specs=[
          pl.BlockSpec((_BR, _H), lambda i: (i, 0)),
          pl.BlockSpec((8, _H), lambda i: (0, 0)),
          pl.BlockSpec((1, _H), lambda i: (0, 0)),
          pl.BlockSpec((1, _H), lambda i: (0, 0)),
          pl.BlockSpec((_H, _H), lambda i: (0, 0)),
          pl.BlockSpec((1, _H), lambda i: (0, 0)),
      ],
      out_specs=pl.BlockSpec((_BR, _H), lambda i: (i, 0)),
      out_shape=jax.ShapeDtypeStruct((_N, _H), jnp.float32),
  )(h1, sums, gamma.reshape(1, _H), beta.reshape(1, _H), W2, b2.reshape(1, _H))


def kernel(x, edge_index, edge_attr, rand_edge, emb0, emb1, emb2,
           W1, b1, gamma, beta, W2, b2, eps):
  x3 = x.reshape(_N, _NC, _HALF)
  x2r = x.reshape(_N * _NC, _HALF)
  src3 = edge_index[0]
  dst3 = edge_index[1]
  a03 = edge_attr[:, 0]
  a13 = edge_attr[:, 1]
  a23 = edge_attr[:, 2]
  rand4 = rand_edge
  embc = (emb0[:, None, None, :] + emb1[None, :, None, :]
          + emb2[None, None, :, :]).reshape(5 * 6 * 2, _BD)
  embc = jnp.concatenate([embc, jnp.zeros((5 * 6 * 2, _R), jnp.float32)],
                         axis=1)
  embc_r = embc.reshape(2 * 5 * 6 * 2, _HALF)
  eps16 = jnp.broadcast_to(eps, (_LANES,)).astype(jnp.float32)

  h = _sc_message_passing(x3, x2r, src3, dst3, a03, a13, a23, rand4, embc_r,
                          eps16)
  h = h.reshape(_N, _H)
  h1, sums = _mlp1(h, W1, b1)
  return _mlp2(h1, sums, gamma, beta, W2, b2)


# speedup vs baseline: 2.5068x; 1.0003x over previous
import functools

import jax
import jax.numpy as jnp
from jax import lax
from jax.experimental import pallas as pl
from jax.experimental.pallas import tpu as pltpu
from jax.experimental.pallas import tpu_sc as plsc

_N = 10000
_E = 160000
_H = 256
_R = 16
_BD = _H - _R
_NC = 2
_NS = 16
_LANES = 16
_HALF = _H // _NC
_VPR = _HALF // _LANES
_EPT = _E // _NS
_KC = 80
_NCHUNK = _EPT // _KC
_RPT = _N // _NS


def _sc_message_passing(x3, x2r, src3, dst3, a03, a13, a23, rand4, embc_r,
                        eps16):
  mesh = plsc.VectorSubcoreMesh(core_axis_name="c", subcore_axis_name="s")

  @functools.partial(
      pl.kernel,
      out_type=jax.ShapeDtypeStruct((_N, _NC, _HALF), jnp.float32),
      mesh=mesh,
      scratch_types=[
          pltpu.VMEM_SHARED((_N, _HALF), jnp.float32),
          pltpu.VMEM((5, _KC), jnp.int32),
          pltpu.VMEM((5, _KC), jnp.int32),
          pltpu.VMEM((2, _KC), jnp.int32),
          pltpu.VMEM((2, _KC), jnp.int32),
          pltpu.VMEM((_KC,), jnp.int32),
          pltpu.VMEM((_KC,), jnp.int32),
          pltpu.VMEM((_KC, _HALF), jnp.float32),
          pltpu.VMEM((_KC, _HALF), jnp.float32),
          pltpu.VMEM((_KC, _R), jnp.float32),
          pltpu.VMEM((_KC, _R), jnp.float32),
          pltpu.VMEM((_LANES,), jnp.float32),
          pltpu.SemaphoreType.DMA,
          pltpu.SemaphoreType.DMA,
          pltpu.SemaphoreType.DMA,
          pltpu.SemaphoreType.DMA,
          pltpu.SemaphoreType.DMA,
          pltpu.SemaphoreType.DMA,
          pltpu.SemaphoreType.DMA,
          pltpu.SemaphoreType.DMA,
          pltpu.SemaphoreType.DMA,
          pltpu.SemaphoreType.DMA,
      ],
  )
  def k(x3h, x2rh, src3h, dst3h, a03h, a13h, a23h, randh, embch, epsh, outh,
        acc, idx0, idx1, gidx0, gidx1, dstb0, dstb1,
        mbuf0, mbuf1, rbuf0, rbuf1, epsv,
        si0, si1, sx0, sx1, se0, se1, sr0, sr1, ss0, ss1):
    c = lax.axis_index("c")
    s = lax.axis_index("s")
    cf = c.astype(jnp.float32)
    pltpu.sync_copy(epsh, epsv)
    e1 = epsv[:] + 1.0

    r0 = s * _RPT

    def init_block(rr, nrows):
      pltpu.sync_copy(x3h.at[pl.ds(rr, nrows), c], mbuf0.at[pl.ds(0, nrows)])

      @plsc.parallel_loop(0, nrows, 1, unroll=4)
      def _(i):
        for v in range(_VPR):
          sl = pl.ds(v * _LANES, _LANES)
          mbuf0[i, sl] = mbuf0[i, sl] * e1

      pltpu.sync_copy(mbuf0.at[pl.ds(0, nrows)], acc.at[pl.ds(rr, nrows)])

    for kk in range(_RPT // _KC):
      init_block(r0 + kk * _KC, _KC)
    init_block(r0 + (_RPT // _KC) * _KC, _RPT % _KC)
    plsc.subcore_barrier()

    e0 = s * _EPT

    def issue_idx(j, ib, si):
      off = e0 + jnp.minimum(j, _NCHUNK - 1) * _KC
      pltpu.async_copy(src3h.at[pl.ds(off, _KC)], ib.at[0], si)
      pltpu.async_copy(dst3h.at[pl.ds(off, _KC)], ib.at[1], si)
      pltpu.async_copy(a03h.at[pl.ds(off, _KC)], ib.at[2], si)
      pltpu.async_copy(a13h.at[pl.ds(off, _KC)], ib.at[3], si)
      pltpu.async_copy(a23h.at[pl.ds(off, _KC)], ib.at[4], si)

    def wait_idx(ib, si):
      for r in range(5):
        pltpu.make_async_copy(src3h.at[pl.ds(0, _KC)], ib.at[r], si).wait()

    def compute_gidx(ib, gb, db):
      for i in range(_KC // _LANES):
        sl = pl.ds(i * _LANES, _LANES)
        gb[0, sl] = ib[0, sl] * 2 + c
        gb[1, sl] = ((ib[2, sl] * 6 + ib[3, sl]) * 2 + ib[4, sl]) * 2 + c
        db[sl] = ib[1, sl]

    def issue_xr(j, gb, mb, rb, sx, sr):
      pltpu.async_copy(x2rh.at[gb.at[0]], mb, sx)
      pltpu.async_copy(randh.at[pl.ds(e0 + j * _KC, _KC)], rb, sr)

    def wait_x(gb, mb, sx):
      pltpu.make_async_copy(x2rh.at[gb.at[0]], mb, sx).wait()

    def issue_emb(gb, mb, se):
      pltpu.async_copy(embch.at[gb.at[1]], mb, se, add=True)

    def wait_emb(gb, mb, se):
      pltpu.make_async_copy(embch.at[gb.at[1]], mb, se).wait()

    def wait_rand(rb, sr):
      pltpu.make_async_copy(randh.at[pl.ds(0, _KC)], rb, sr).wait()

    def relu_scatter(mb, rb, db, ss):
      @plsc.parallel_loop(0, _KC, 1, unroll=4)
      def _(r):
        for v in range(_VPR):
          sl = pl.ds(v * _LANES, _LANES)
          val = mb[r, sl]
          if v == _VPR - 1:
            val = val + rb[r, :] * cf
          mb[r, sl] = jnp.maximum(val, 0.0)

      pltpu.async_copy(mb, acc.at[db], ss, add=True)

    def wait_scatter(mb, db, ss):
      pltpu.make_async_copy(mb, acc.at[db], ss).wait()

    P0 = (idx0, gidx0, dstb0, mbuf0, rbuf0, si0, sx0, se0, sr0, ss0)
    P1 = (idx1, gidx1, dstb1, mbuf1, rbuf1, si1, sx1, se1, sr1, ss1)

    def half(j, B, Bo, wait_prev_scatter):
      (ib, gb, db, mb, rb, si, sx, se, sr, ss) = B
      (ibo, gbo, dbo, mbo, rbo, sio, sxo, seo, sro, sso) = Bo
      wait_idx(ibo, sio)
      if wait_prev_scatter is True:
        wait_scatter(mbo, dbo, sso)
      elif wait_prev_scatter is not False:
        @pl.when(wait_prev_scatter)
        def _():
          wait_scatter(mbo, dbo, sso)
      compute_gidx(ibo, gbo, dbo)
      issue_xr(j + 1, gbo, mbo, rbo, sxo, sro)
      issue_idx(j + 2, ib, si)
      wait_emb(gb, mb, se)
      wait_rand(rb, sr)
      relu_scatter(mb, rb, db, ss)
      wait_x(gbo, mbo, sxo)
      issue_emb(gbo, mbo, seo)

    pltpu.sync_copy(src3h.at[pl.ds(e0, _KC)], idx0.at[0])
    pltpu.sync_copy(dst3h.at[pl.ds(e0, _KC)], idx0.at[1])
    pltpu.sync_copy(a03h.at[pl.ds(e0, _KC)], idx0.at[2])
    pltpu.sync_copy(a13h.at[pl.ds(e0, _KC)], idx0.at[3])
    pltpu.sync_copy(a23h.at[pl.ds(e0, _KC)], idx0.at[4])
    compute_gidx(idx0, gidx0, dstb0)
    issue_xr(0, gidx0, mbuf0, rbuf0, sx0, sr0)
    issue_idx(1, idx1, si1)
    wait_x(gidx0, mbuf0, sx0)
    issue_emb(gidx0, mbuf0, se0)

    def pair(g, _):
      j0 = g * 2
      half(j0, P0, P1, jnp.greater(g, 0))
      half(j0 + 1, P1, P0, True)
      return 0

    lax.fori_loop(0, (_NCHUNK - 1) // 2, pair, 0)
    wait_emb(gidx0, mbuf0, se0)
    wait_rand(rbuf0, sr0)
    relu_scatter(mbuf0, rbuf0, dstb0, ss0)
    wait_scatter(mbuf0, dstb0, ss0)
    wait_scatter(mbuf1, dstb1, ss1)
    wait_idx(idx1, si1)

    plsc.subcore_barrier()

    pltpu.sync_copy(acc.at[pl.ds(r0, _RPT)], outh.at[pl.ds(r0, _RPT), c])

  return k(x3, x2r, src3, dst3, a03, a13, a23, rand4, embc_r, eps16)


_NB = 10
_BR = _N // _NB


def _mlp1(h, W1, b1):
  def body(h_ref, w_ref, b_ref, h1_ref, sums_ref, accs):
    i = pl.program_id(0)
    h1 = jnp.dot(h_ref[:], w_ref[:], preferred_element_type=jnp.float32)
    h1 = h1 + b_ref[:]
    h1_ref[:] = h1

    @pl.when(i == 0)
    def _():
      accs[:] = jnp.zeros_like(accs)

    accs[0:1, :] = accs[0:1, :] + jnp.sum(h1, axis=0, keepdims=True)
    accs[1:2, :] = accs[1:2, :] + jnp.sum(h1 * h1, axis=0, keepdims=True)
    sums_ref[:] = accs[:]

  return pl.pallas_call(
      body,
      grid=(_NB,),
      in_specs=[
          pl.BlockSpec((_BR, _H), lambda i: (i, 0)),
          pl.BlockSpec((_H, _H), lambda i: (0, 0)),
          pl.BlockSpec((1, _H), lambda i: (0, 0)),
      ],
      out_specs=[
          pl.BlockSpec((_BR, _H), lambda i: (i, 0)),
          pl.BlockSpec((8, _H), lambda i: (0, 0)),
      ],
      out_shape=[
          jax.ShapeDtypeStruct((_N, _H), jnp.float32),
          jax.ShapeDtypeStruct((8, _H), jnp.float32),
      ],
      scratch_shapes=[pltpu.VMEM((8, _H), jnp.float32)],
  )(h, W1, b1.reshape(1, _H))


def _mlp2(h1, sums, gamma, beta, W2, b2):
  def body(h1_ref, sums_ref, g_ref, be_ref, w_ref, b_ref, o_ref):
    mu = sums_ref[0:1, :] / _N
    ms = sums_ref[1:2, :] / _N
    var = ms - mu * mu
    inv = lax.rsqrt(var + 1e-5)
    a = (h1_ref[:] - mu) * (inv * g_ref[:]) + be_ref[:]
    a = jnp.maximum(a, 0.0)
    o_ref[:] = jnp.dot(a, w_ref[:], preferred_element_type=jnp.float32) + b_ref[:]

  return pl.pallas_call(
      body,
      grid=(_NB,),
      in_specs=[
          pl.BlockSpec((_BR, _H), lambda i: (i, 0)),
          pl.BlockSpec((8, _H), lambda i: (0, 0)),
          pl.BlockSpec((1, _H), lambda i: (0, 0)),
          pl.BlockSpec((1, _H), lambda i: (0, 0)),
          pl.BlockSpec((_H, _H), lambda i: (0, 0)),
          pl.BlockSpec((1, _H), lambda i: (0, 0)),
      ],
      out_specs=pl.BlockSpec((_BR, _H), lambda i: (i, 0)),
      out_shape=jax.ShapeDtypeStruct((_N, _H), jnp.float32),
  )(h1, sums, gamma.reshape(1, _H), beta.reshape(1, _H), W2, b2.reshape(1, _H))


def kernel(x, edge_index, edge_attr, rand_edge, emb0, emb1, emb2,
           W1, b1, gamma, beta, W2, b2, eps):
  x3 = x.reshape(_N, _NC, _HALF)
  x2r = x.reshape(_N * _NC, _HALF)
  src3 = edge_index[0]
  dst3 = edge_index[1]
  a03 = edge_attr[:, 0]
  a13 = edge_attr[:, 1]
  a23 = edge_attr[:, 2]
  rand4 = rand_edge
  embc = (emb0[:, None, None, :] + emb1[None, :, None, :]
          + emb2[None, None, :, :]).reshape(5 * 6 * 2, _BD)
  embc = jnp.concatenate([embc, jnp.zeros((5 * 6 * 2, _R), jnp.float32)],
                         axis=1)
  embc_r = embc.reshape(2 * 5 * 6 * 2, _HALF)
  eps16 = jnp.broadcast_to(eps, (_LANES,)).astype(jnp.float32)

  h = _sc_message_passing(x3, x2r, src3, dst3, a03, a13, a23, rand4, embc_r,
                          eps16)
  h = h.reshape(_N, _H)
  h1, sums = _mlp1(h, W1, b1)
  return _mlp2(h1, sums, gamma, beta, W2, b2)


# speedup vs baseline: 6.8886x; 2.7480x over previous
import functools

import jax
import jax.numpy as jnp
from jax import lax
from jax.experimental import pallas as pl
from jax.experimental.pallas import tpu as pltpu
from jax.experimental.pallas import tpu_sc as plsc

_N = 10000
_E = 160000
_H = 256
_R = 16
_BD = _H - _R
_NC = 2
_NS = 16
_LANES = 16
_HALF = _H // _NC
_VPR = _HALF // _LANES
_EPT = _E // _NS
_KC = 80
_NCHUNK = _EPT // _KC
_RPT = _N // _NS


def _sc_message_passing(x3, x2r, src3, dst3, a03, a13, a23, rand4, embc_r,
                        eps16):
  mesh = plsc.VectorSubcoreMesh(core_axis_name="c", subcore_axis_name="s")

  @functools.partial(
      pl.kernel,
      out_type=jax.ShapeDtypeStruct((_N, _NC, _HALF), jnp.float32),
      mesh=mesh,
      scratch_types=[
          pltpu.VMEM_SHARED((_N, _HALF), jnp.float32),
          pltpu.VMEM_SHARED((2 * 5 * 6 * 2, _HALF), jnp.float32),
          pltpu.VMEM((5, _KC), jnp.int32),
          pltpu.VMEM((5, _KC), jnp.int32),
          pltpu.VMEM((2, _KC), jnp.int32),
          pltpu.VMEM((2, _KC), jnp.int32),
          pltpu.VMEM((_KC,), jnp.int32),
          pltpu.VMEM((_KC,), jnp.int32),
          pltpu.VMEM((_KC, _HALF), jnp.float32),
          pltpu.VMEM((_KC, _HALF), jnp.float32),
          pltpu.VMEM((_KC, _R), jnp.float32),
          pltpu.VMEM((_KC, _R), jnp.float32),
          pltpu.VMEM((_LANES,), jnp.float32),
          pltpu.SemaphoreType.DMA,
          pltpu.SemaphoreType.DMA,
          pltpu.SemaphoreType.DMA,
          pltpu.SemaphoreType.DMA,
          pltpu.SemaphoreType.DMA,
          pltpu.SemaphoreType.DMA,
          pltpu.SemaphoreType.DMA,
          pltpu.SemaphoreType.DMA,
          pltpu.SemaphoreType.DMA,
          pltpu.SemaphoreType.DMA,
      ],
  )
  def k(x3h, x2rh, src3h, dst3h, a03h, a13h, a23h, randh, embch, epsh, outh,
        acc, embs, idx0, idx1, gidx0, gidx1, dstb0, dstb1,
        mbuf0, mbuf1, rbuf0, rbuf1, epsv,
        si0, si1, sx0, sx1, se0, se1, sr0, sr1, ss0, ss1):
    c = lax.axis_index("c")
    s = lax.axis_index("s")
    cf = c.astype(jnp.float32)
    pltpu.sync_copy(epsh, epsv)
    e1 = epsv[:] + 1.0

    @pl.when(s == 0)
    def _():
      pltpu.sync_copy(embch.at[pl.ds(0, 64)], mbuf0.at[pl.ds(0, 64)])
      pltpu.sync_copy(mbuf0.at[pl.ds(0, 64)], embs.at[pl.ds(0, 64)])
      pltpu.sync_copy(embch.at[pl.ds(64, 56)], mbuf0.at[pl.ds(0, 56)])
      pltpu.sync_copy(mbuf0.at[pl.ds(0, 56)], embs.at[pl.ds(64, 56)])

    r0 = s * _RPT

    def init_block(rr, nrows):
      pltpu.sync_copy(x3h.at[pl.ds(rr, nrows), c], mbuf0.at[pl.ds(0, nrows)])

      @plsc.parallel_loop(0, nrows, 1, unroll=4)
      def _(i):
        for v in range(_VPR):
          sl = pl.ds(v * _LANES, _LANES)
          mbuf0[i, sl] = mbuf0[i, sl] * e1

      pltpu.sync_copy(mbuf0.at[pl.ds(0, nrows)], acc.at[pl.ds(rr, nrows)])

    for kk in range(_RPT // _KC):
      init_block(r0 + kk * _KC, _KC)
    init_block(r0 + (_RPT // _KC) * _KC, _RPT % _KC)
    plsc.subcore_barrier()

    e0 = s * _EPT

    def issue_idx(j, ib, si):
      off = e0 + jnp.minimum(j, _NCHUNK - 1) * _KC
      pltpu.async_copy(src3h.at[pl.ds(off, _KC)], ib.at[0], si)
      pltpu.async_copy(dst3h.at[pl.ds(off, _KC)], ib.at[1], si)
      pltpu.async_copy(a03h.at[pl.ds(off, _KC)], ib.at[2], si)
      pltpu.async_copy(a13h.at[pl.ds(off, _KC)], ib.at[3], si)
      pltpu.async_copy(a23h.at[pl.ds(off, _KC)], ib.at[4], si)

    def wait_idx(ib, si):
      for r in range(5):
        pltpu.make_async_copy(src3h.at[pl.ds(0, _KC)], ib.at[r], si).wait()

    def compute_gidx(ib, gb, db):
      for i in range(_KC // _LANES):
        sl = pl.ds(i * _LANES, _LANES)
        gb[0, sl] = ib[0, sl] * 2 + c
        gb[1, sl] = ((ib[2, sl] * 6 + ib[3, sl]) * 2 + ib[4, sl]) * 2 + c
        db[sl] = ib[1, sl]

    def issue_xr(j, gb, mb, rb, sx, sr):
      pltpu.async_copy(x2rh.at[gb.at[0]], mb, sx)
      pltpu.async_copy(randh.at[pl.ds(e0 + j * _KC, _KC)], rb, sr)

    def wait_x(gb, mb, sx):
      pltpu.make_async_copy(x2rh.at[gb.at[0]], mb, sx).wait()

    def issue_emb(gb, mb, se):
      pltpu.async_copy(embs.at[gb.at[1]], mb, se, add=True)

    def wait_emb(gb, mb, se):
      pltpu.make_async_copy(embs.at[gb.at[1]], mb, se).wait()

    def wait_rand(rb, sr):
      pltpu.make_async_copy(randh.at[pl.ds(0, _KC)], rb, sr).wait()

    def relu_scatter(mb, rb, db, ss):
      @plsc.parallel_loop(0, _KC, 1, unroll=4)
      def _(r):
        for v in range(_VPR):
          sl = pl.ds(v * _LANES, _LANES)
          val = mb[r, sl]
          if v == _VPR - 1:
            val = val + rb[r, :] * cf
          mb[r, sl] = jnp.maximum(val, 0.0)

      pltpu.async_copy(mb, acc.at[db], ss, add=True)

    def wait_scatter(mb, db, ss):
      pltpu.make_async_copy(mb, acc.at[db], ss).wait()

    P0 = (idx0, gidx0, dstb0, mbuf0, rbuf0, si0, sx0, se0, sr0, ss0)
    P1 = (idx1, gidx1, dstb1, mbuf1, rbuf1, si1, sx1, se1, sr1, ss1)

    def half(j, B, Bo, wait_prev_scatter):
      (ib, gb, db, mb, rb, si, sx, se, sr, ss) = B
      (ibo, gbo, dbo, mbo, rbo, sio, sxo, seo, sro, sso) = Bo
      wait_idx(ibo, sio)
      if wait_prev_scatter is True:
        wait_scatter(mbo, dbo, sso)
      elif wait_prev_scatter is not False:
        @pl.when(wait_prev_scatter)
        def _():
          wait_scatter(mbo, dbo, sso)
      compute_gidx(ibo, gbo, dbo)
      issue_xr(j + 1, gbo, mbo, rbo, sxo, sro)
      issue_idx(j + 2, ib, si)
      wait_emb(gb, mb, se)
      wait_rand(rb, sr)
      relu_scatter(mb, rb, db, ss)
      wait_x(gbo, mbo, sxo)
      issue_emb(gbo, mbo, seo)

    pltpu.sync_copy(src3h.at[pl.ds(e0, _KC)], idx0.at[0])
    pltpu.sync_copy(dst3h.at[pl.ds(e0, _KC)], idx0.at[1])
    pltpu.sync_copy(a03h.at[pl.ds(e0, _KC)], idx0.at[2])
    pltpu.sync_copy(a13h.at[pl.ds(e0, _KC)], idx0.at[3])
    pltpu.sync_copy(a23h.at[pl.ds(e0, _KC)], idx0.at[4])
    compute_gidx(idx0, gidx0, dstb0)
    issue_xr(0, gidx0, mbuf0, rbuf0, sx0, sr0)
    issue_idx(1, idx1, si1)
    wait_x(gidx0, mbuf0, sx0)
    issue_emb(gidx0, mbuf0, se0)

    def pair(g, _):
      j0 = g * 2
      half(j0, P0, P1, jnp.greater(g, 0))
      half(j0 + 1, P1, P0, True)
      return 0

    lax.fori_loop(0, (_NCHUNK - 1) // 2, pair, 0)
    wait_emb(gidx0, mbuf0, se0)
    wait_rand(rbuf0, sr0)
    relu_scatter(mbuf0, rbuf0, dstb0, ss0)
    wait_scatter(mbuf0, dstb0, ss0)
    wait_scatter(mbuf1, dstb1, ss1)
    wait_idx(idx1, si1)

    plsc.subcore_barrier()

    pltpu.sync_copy(acc.at[pl.ds(r0, _RPT)], outh.at[pl.ds(r0, _RPT), c])

  return k(x3, x2r, src3, dst3, a03, a13, a23, rand4, embc_r, eps16)


_NB = 10
_BR = _N // _NB


def _mlp1(h, W1, b1):
  def body(h_ref, w_ref, b_ref, h1_ref, sums_ref, accs):
    i = pl.program_id(0)
    h1 = jnp.dot(h_ref[:], w_ref[:], preferred_element_type=jnp.float32)
    h1 = h1 + b_ref[:]
    h1_ref[:] = h1

    @pl.when(i == 0)
    def _():
      accs[:] = jnp.zeros_like(accs)

    accs[0:1, :] = accs[0:1, :] + jnp.sum(h1, axis=0, keepdims=True)
    accs[1:2, :] = accs[1:2, :] + jnp.sum(h1 * h1, axis=0, keepdims=True)
    sums_ref[:] = accs[:]

  return pl.pallas_call(
      body,
      grid=(_NB,),
      in_specs=[
          pl.BlockSpec((_BR, _H), lambda i: (i, 0)),
          pl.BlockSpec((_H, _H), lambda i: (0, 0)),
          pl.BlockSpec((1, _H), lambda i: (0, 0)),
      ],
      out_specs=[
          pl.BlockSpec((_BR, _H), lambda i: (i, 0)),
          pl.BlockSpec((8, _H), lambda i: (0, 0)),
      ],
      out_shape=[
          jax.ShapeDtypeStruct((_N, _H), jnp.float32),
          jax.ShapeDtypeStruct((8, _H), jnp.float32),
      ],
      scratch_shapes=[pltpu.VMEM((8, _H), jnp.float32)],
  )(h, W1, b1.reshape(1, _H))


def _mlp2(h1, sums, gamma, beta, W2, b2):
  def body(h1_ref, sums_ref, g_ref, be_ref, w_ref, b_ref, o_ref):
    mu = sums_ref[0:1, :] / _N
    ms = sums_ref[1:2, :] / _N
    var = ms - mu * mu
    inv = lax.rsqrt(var + 1e-5)
    a = (h1_ref[:] - mu) * (inv * g_ref[:]) + be_ref[:]
    a = jnp.maximum(a, 0.0)
    o_ref[:] = jnp.dot(a, w_ref[:], preferred_element_type=jnp.float32) + b_ref[:]

  return pl.pallas_call(
      body,
      grid=(_NB,),
      in_specs=[
          pl.BlockSpec((_BR, _H), lambda i: (i, 0)),
          pl.BlockSpec((8, _H), lambda i: (0, 0)),
          pl.BlockSpec((1, _H), lambda i: (0, 0)),
          pl.BlockSpec((1, _H), lambda i: (0, 0)),
          pl.BlockSpec((_H, _H), lambda i: (0, 0)),
          pl.BlockSpec((1, _H), lambda i: (0, 0)),
      ],
      out_specs=pl.BlockSpec((_BR, _H), lambda i: (i, 0)),
      out_shape=jax.ShapeDtypeStruct((_N, _H), jnp.float32),
  )(h1, sums, gamma.reshape(1, _H), beta.reshape(1, _H), W2, b2.reshape(1, _H))


def kernel(x, edge_index, edge_attr, rand_edge, emb0, emb1, emb2,
           W1, b1, gamma, beta, W2, b2, eps):
  x3 = x.reshape(_N, _NC, _HALF)
  x2r = x.reshape(_N * _NC, _HALF)
  src3 = edge_index[0]
  dst3 = edge_index[1]
  a03 = edge_attr[:, 0]
  a13 = edge_attr[:, 1]
  a23 = edge_attr[:, 2]
  rand4 = rand_edge
  embc = (emb0[:, None, None, :] + emb1[None, :, None, :]
          + emb2[None, None, :, :]).reshape(5 * 6 * 2, _BD)
  embc = jnp.concatenate([embc, jnp.zeros((5 * 6 * 2, _R), jnp.float32)],
                         axis=1)
  embc_r = embc.reshape(2 * 5 * 6 * 2, _HALF)
  eps16 = jnp.broadcast_to(eps, (_LANES,)).astype(jnp.float32)

  h = _sc_message_passing(x3, x2r, src3, dst3, a03, a13, a23, rand4, embc_r,
                          eps16)
  h = h.reshape(_N, _H)
  h1, sums = _mlp1(h, W1, b1)
  return _mlp2(h1, sums, gamma, beta, W2, b2)


# speedup vs baseline: 7.0626x; 1.0252x over previous
import functools

import jax
import jax.numpy as jnp
from jax import lax
from jax.experimental import pallas as pl
from jax.experimental.pallas import tpu as pltpu
from jax.experimental.pallas import tpu_sc as plsc

_N = 10000
_E = 160000
_H = 256
_R = 16
_BD = _H - _R
_NC = 2
_NS = 16
_LANES = 16
_HALF = _H // _NC
_VPR = _HALF // _LANES
_EPT = _E // _NS
_KC = 80
_NCHUNK = _EPT // _KC
_RPT = _N // _NS


def _sc_message_passing(x3, x2r, src3, dst3, a03, rand4, embc_r, eps16):
  mesh = plsc.VectorSubcoreMesh(core_axis_name="c", subcore_axis_name="s")

  @functools.partial(
      pl.kernel,
      out_type=jax.ShapeDtypeStruct((_N, _NC, _HALF), jnp.float32),
      mesh=mesh,
      scratch_types=[
          pltpu.VMEM_SHARED((_N, _HALF), jnp.float32),
          pltpu.VMEM_SHARED((2 * 5 * 6 * 2, _HALF), jnp.float32),
          pltpu.VMEM((3, _KC), jnp.int32),
          pltpu.VMEM((3, _KC), jnp.int32),
          pltpu.VMEM((2, _KC), jnp.int32),
          pltpu.VMEM((2, _KC), jnp.int32),
          pltpu.VMEM((_KC,), jnp.int32),
          pltpu.VMEM((_KC,), jnp.int32),
          pltpu.VMEM((_KC, _HALF), jnp.float32),
          pltpu.VMEM((_KC, _HALF), jnp.float32),
          pltpu.VMEM((_KC, _R), jnp.float32),
          pltpu.VMEM((_KC, _R), jnp.float32),
          pltpu.VMEM((_LANES,), jnp.float32),
          pltpu.SemaphoreType.DMA,
          pltpu.SemaphoreType.DMA,
          pltpu.SemaphoreType.DMA,
          pltpu.SemaphoreType.DMA,
          pltpu.SemaphoreType.DMA,
          pltpu.SemaphoreType.DMA,
          pltpu.SemaphoreType.DMA,
          pltpu.SemaphoreType.DMA,
          pltpu.SemaphoreType.DMA,
          pltpu.SemaphoreType.DMA,
      ],
  )
  def k(x3h, x2rh, src3h, dst3h, cidxh, randh, embch, epsh, outh,
        acc, embs, idx0, idx1, gidx0, gidx1, dstb0, dstb1,
        mbuf0, mbuf1, rbuf0, rbuf1, epsv,
        si0, si1, sx0, sx1, se0, se1, sr0, sr1, ss0, ss1):
    c = lax.axis_index("c")
    s = lax.axis_index("s")
    cf = c.astype(jnp.float32)
    pltpu.sync_copy(epsh, epsv)
    e1 = epsv[:] + 1.0

    @pl.when(s == 0)
    def _():
      pltpu.sync_copy(embch.at[pl.ds(0, 64)], mbuf0.at[pl.ds(0, 64)])
      pltpu.sync_copy(mbuf0.at[pl.ds(0, 64)], embs.at[pl.ds(0, 64)])
      pltpu.sync_copy(embch.at[pl.ds(64, 56)], mbuf0.at[pl.ds(0, 56)])
      pltpu.sync_copy(mbuf0.at[pl.ds(0, 56)], embs.at[pl.ds(64, 56)])

    r0 = s * _RPT

    def init_block(rr, nrows):
      pltpu.sync_copy(x3h.at[pl.ds(rr, nrows), c], mbuf0.at[pl.ds(0, nrows)])

      @plsc.parallel_loop(0, nrows, 1, unroll=4)
      def _(i):
        for v in range(_VPR):
          sl = pl.ds(v * _LANES, _LANES)
          mbuf0[i, sl] = mbuf0[i, sl] * e1

      pltpu.sync_copy(mbuf0.at[pl.ds(0, nrows)], acc.at[pl.ds(rr, nrows)])

    for kk in range(_RPT // _KC):
      init_block(r0 + kk * _KC, _KC)
    init_block(r0 + (_RPT // _KC) * _KC, _RPT % _KC)
    plsc.subcore_barrier()

    e0 = s * _EPT

    def issue_idx(j, ib, si):
      off = e0 + jnp.minimum(j, _NCHUNK - 1) * _KC
      pltpu.async_copy(src3h.at[pl.ds(off, _KC)], ib.at[0], si)
      pltpu.async_copy(dst3h.at[pl.ds(off, _KC)], ib.at[1], si)
      pltpu.async_copy(cidxh.at[pl.ds(off, _KC)], ib.at[2], si)

    def wait_idx(ib, si):
      for r in range(3):
        pltpu.make_async_copy(src3h.at[pl.ds(0, _KC)], ib.at[r], si).wait()

    def compute_gidx(ib, gb, db):
      for i in range(_KC // _LANES):
        sl = pl.ds(i * _LANES, _LANES)
        gb[0, sl] = ib[0, sl] * 2 + c
        gb[1, sl] = ib[2, sl] * 2 + c
        db[sl] = ib[1, sl]

    def issue_xr(j, gb, mb, rb, sx, sr):
      pltpu.async_copy(x2rh.at[gb.at[0]], mb, sx)
      pltpu.async_copy(randh.at[pl.ds(e0 + j * _KC, _KC)], rb, sr)

    def wait_x(gb, mb, sx):
      pltpu.make_async_copy(x2rh.at[gb.at[0]], mb, sx).wait()

    def issue_emb(gb, mb, se):
      pltpu.async_copy(embs.at[gb.at[1]], mb, se, add=True)

    def wait_emb(gb, mb, se):
      pltpu.make_async_copy(embs.at[gb.at[1]], mb, se).wait()

    def wait_rand(rb, sr):
      pltpu.make_async_copy(randh.at[pl.ds(0, _KC)], rb, sr).wait()

    def relu_scatter(mb, rb, db, ss):
      @plsc.parallel_loop(0, _KC, 1, unroll=4)
      def _(r):
        for v in range(_VPR):
          sl = pl.ds(v * _LANES, _LANES)
          val = mb[r, sl]
          if v == _VPR - 1:
            val = val + rb[r, :] * cf
          mb[r, sl] = jnp.maximum(val, 0.0)

      pltpu.async_copy(mb, acc.at[db], ss, add=True)

    def wait_scatter(mb, db, ss):
      pltpu.make_async_copy(mb, acc.at[db], ss).wait()

    P0 = (idx0, gidx0, dstb0, mbuf0, rbuf0, si0, sx0, se0, sr0, ss0)
    P1 = (idx1, gidx1, dstb1, mbuf1, rbuf1, si1, sx1, se1, sr1, ss1)

    def half(j, B, Bo, wait_prev_scatter):
      (ib, gb, db, mb, rb, si, sx, se, sr, ss) = B
      (ibo, gbo, dbo, mbo, rbo, sio, sxo, seo, sro, sso) = Bo
      wait_idx(ibo, sio)
      if wait_prev_scatter is True:
        wait_scatter(mbo, dbo, sso)
      elif wait_prev_scatter is not False:
        @pl.when(wait_prev_scatter)
        def _():
          wait_scatter(mbo, dbo, sso)
      compute_gidx(ibo, gbo, dbo)
      issue_xr(j + 1, gbo, mbo, rbo, sxo, sro)
      issue_idx(j + 2, ib, si)
      wait_emb(gb, mb, se)
      wait_rand(rb, sr)
      relu_scatter(mb, rb, db, ss)
      wait_x(gbo, mbo, sxo)
      issue_emb(gbo, mbo, seo)

    pltpu.sync_copy(src3h.at[pl.ds(e0, _KC)], idx0.at[0])
    pltpu.sync_copy(dst3h.at[pl.ds(e0, _KC)], idx0.at[1])
    pltpu.sync_copy(cidxh.at[pl.ds(e0, _KC)], idx0.at[2])
    compute_gidx(idx0, gidx0, dstb0)
    issue_xr(0, gidx0, mbuf0, rbuf0, sx0, sr0)
    issue_idx(1, idx1, si1)
    wait_x(gidx0, mbuf0, sx0)
    issue_emb(gidx0, mbuf0, se0)

    def pair(g, _):
      j0 = g * 2
      half(j0, P0, P1, jnp.greater(g, 0))
      half(j0 + 1, P1, P0, True)
      return 0

    lax.fori_loop(0, (_NCHUNK - 1) // 2, pair, 0)
    wait_emb(gidx0, mbuf0, se0)
    wait_rand(rbuf0, sr0)
    relu_scatter(mbuf0, rbuf0, dstb0, ss0)
    wait_scatter(mbuf0, dstb0, ss0)
    wait_scatter(mbuf1, dstb1, ss1)
    wait_idx(idx1, si1)

    plsc.subcore_barrier()

    pltpu.sync_copy(acc.at[pl.ds(r0, _RPT)], outh.at[pl.ds(r0, _RPT), c])

  return k(x3, x2r, src3, dst3, a03, rand4, embc_r, eps16)


_NB = 10
_BR = _N // _NB


def _mlp(h, W1, b1, gamma, beta, W2, b2):
  def body(h_ref, w1_ref, b1_ref, g_ref, be_ref, w2_ref, b2_ref, o_ref,
           h1s, accs):
    p = pl.program_id(0)
    i = pl.program_id(1)

    @pl.when(p == 0)
    def _():
      h1 = jnp.dot(h_ref[:], w1_ref[:], preferred_element_type=jnp.float32)
      h1 = h1 + b1_ref[:]
      h1s[pl.ds(i * _BR, _BR), :] = h1

      @pl.when(i == 0)
      def _():
        accs[:] = jnp.zeros_like(accs)

      accs[0:1, :] = accs[0:1, :] + jnp.sum(h1, axis=0, keepdims=True)
      accs[1:2, :] = accs[1:2, :] + jnp.sum(h1 * h1, axis=0, keepdims=True)

    @pl.when(p == 1)
    def _():
      mu = accs[0:1, :] / _N
      var = accs[1:2, :] / _N - mu * mu
      inv = lax.rsqrt(var + 1e-5)
      a = (h1s[pl.ds(i * _BR, _BR), :] - mu) * (inv * g_ref[:]) + be_ref[:]
      a = jnp.maximum(a, 0.0)
      o_ref[:] = jnp.dot(a, w2_ref[:],
                         preferred_element_type=jnp.float32) + b2_ref[:]

  cst = lambda p, i: (0, 0)
  return pl.pallas_call(
      body,
      grid=(2, _NB),
      in_specs=[
          pl.BlockSpec((_BR, _H), lambda p, i: (i * (1 - p), 0)),
          pl.BlockSpec((_H, _H), cst),
          pl.BlockSpec((1, _H), cst),
          pl.BlockSpec((1, _H), cst),
          pl.BlockSpec((1, _H), cst),
          pl.BlockSpec((_H, _H), cst),
          pl.BlockSpec((1, _H), cst),
      ],
      out_specs=pl.BlockSpec((_BR, _H), lambda p, i: (i * p, 0)),
      out_shape=jax.ShapeDtypeStruct((_N, _H), jnp.float32),
      scratch_shapes=[
          pltpu.VMEM((_N, _H), jnp.float32),
          pltpu.VMEM((8, _H), jnp.float32),
      ],
  )(h, W1, b1.reshape(1, _H), gamma.reshape(1, _H), beta.reshape(1, _H),
    W2, b2.reshape(1, _H))


def kernel(x, edge_index, edge_attr, rand_edge, emb0, emb1, emb2,
           W1, b1, gamma, beta, W2, b2, eps):
  x3 = x.reshape(_N, _NC, _HALF)
  x2r = x.reshape(_N * _NC, _HALF)
  src3 = edge_index[0]
  dst3 = edge_index[1]
  cidx = (edge_attr[:, 0] * 6 + edge_attr[:, 1]) * 2 + edge_attr[:, 2]
  rand4 = rand_edge
  embc = (emb0[:, None, None, :] + emb1[None, :, None, :]
          + emb2[None, None, :, :]).reshape(5 * 6 * 2, _BD)
  embc = jnp.concatenate([embc, jnp.zeros((5 * 6 * 2, _R), jnp.float32)],
                         axis=1)
  embc_r = embc.reshape(2 * 5 * 6 * 2, _HALF)
  eps16 = jnp.broadcast_to(eps, (_LANES,)).astype(jnp.float32)

  h = _sc_message_passing(x3, x2r, src3, dst3, cidx, rand4, embc_r, eps16)
  h = h.reshape(_N, _H)
  return _mlp(h, W1, b1, gamma, beta, W2, b2)


# speedup vs baseline: 7.1643x; 1.0144x over previous
import functools

import jax
import jax.numpy as jnp
from jax import lax
from jax.experimental import pallas as pl
from jax.experimental.pallas import tpu as pltpu
from jax.experimental.pallas import tpu_sc as plsc

_N = 10000
_E = 160000
_H = 256
_R = 16
_BD = _H - _R
_NC = 2
_NS = 16
_LANES = 16
_HALF = _H // _NC
_VPR = _HALF // _LANES
_EPT = _E // _NS
_KC = 80
_NCHUNK = _EPT // _KC
_RPT = _N // _NS


def _sc_message_passing(x3, x2r, src3, dst3, a03, rand4, embc_r, eps16):
  mesh = plsc.VectorSubcoreMesh(core_axis_name="c", subcore_axis_name="s")

  @functools.partial(
      pl.kernel,
      out_type=jax.ShapeDtypeStruct((_N, _NC, _HALF), jnp.float32),
      mesh=mesh,
      scratch_types=[
          pltpu.VMEM_SHARED((_N, _HALF), jnp.float32),
          pltpu.VMEM_SHARED((2 * 5 * 6 * 2, _HALF), jnp.float32),
          pltpu.VMEM((3, _KC), jnp.int32),
          pltpu.VMEM((3, _KC), jnp.int32),
          pltpu.VMEM((2, _KC), jnp.int32),
          pltpu.VMEM((2, _KC), jnp.int32),
          pltpu.VMEM((_KC,), jnp.int32),
          pltpu.VMEM((_KC,), jnp.int32),
          pltpu.VMEM((_KC, _HALF), jnp.float32),
          pltpu.VMEM((_KC, _HALF), jnp.float32),
          pltpu.VMEM((_KC, _R), jnp.float32),
          pltpu.VMEM((_KC, _R), jnp.float32),
          pltpu.VMEM((_LANES,), jnp.float32),
          pltpu.SemaphoreType.DMA,
          pltpu.SemaphoreType.DMA,
          pltpu.SemaphoreType.DMA,
          pltpu.SemaphoreType.DMA,
          pltpu.SemaphoreType.DMA,
          pltpu.SemaphoreType.DMA,
          pltpu.SemaphoreType.DMA,
          pltpu.SemaphoreType.DMA,
          pltpu.SemaphoreType.DMA,
          pltpu.SemaphoreType.DMA,
      ],
  )
  def k(x3h, x2rh, src3h, dst3h, cidxh, randh, embch, epsh, outh,
        acc, embs, idx0, idx1, gidx0, gidx1, dstb0, dstb1,
        mbuf0, mbuf1, rbuf0, rbuf1, epsv,
        si0, si1, sx0, sx1, se0, se1, sr0, sr1, ss0, ss1):
    c = lax.axis_index("c")
    s = lax.axis_index("s")
    cf = c.astype(jnp.float32)
    pltpu.sync_copy(epsh, epsv)
    e1 = epsv[:] + 1.0

    @pl.when(s == 0)
    def _():
      pltpu.sync_copy(embch.at[pl.ds(0, 64)], mbuf0.at[pl.ds(0, 64)])
      pltpu.sync_copy(mbuf0.at[pl.ds(0, 64)], embs.at[pl.ds(0, 64)])
      pltpu.sync_copy(embch.at[pl.ds(64, 56)], mbuf0.at[pl.ds(0, 56)])
      pltpu.sync_copy(mbuf0.at[pl.ds(0, 56)], embs.at[pl.ds(64, 56)])

    r0 = s * _RPT

    def init_block(rr, nrows):
      pltpu.sync_copy(x3h.at[pl.ds(rr, nrows), c], mbuf0.at[pl.ds(0, nrows)])

      @plsc.parallel_loop(0, nrows, 1, unroll=4)
      def _(i):
        for v in range(_VPR):
          sl = pl.ds(v * _LANES, _LANES)
          mbuf0[i, sl] = mbuf0[i, sl] * e1

      pltpu.sync_copy(mbuf0.at[pl.ds(0, nrows)], acc.at[pl.ds(rr, nrows)])

    for kk in range(_RPT // _KC):
      init_block(r0 + kk * _KC, _KC)
    init_block(r0 + (_RPT // _KC) * _KC, _RPT % _KC)
    plsc.subcore_barrier()

    e0 = s * _EPT

    def issue_idx(j, ib, si):
      off = e0 + jnp.minimum(j, _NCHUNK - 1) * _KC
      pltpu.async_copy(src3h.at[pl.ds(off, _KC)], ib.at[0], si)
      pltpu.async_copy(dst3h.at[pl.ds(off, _KC)], ib.at[1], si)
      pltpu.async_copy(cidxh.at[pl.ds(off, _KC)], ib.at[2], si)

    def wait_idx(ib, si):
      for r in range(3):
        pltpu.make_async_copy(src3h.at[pl.ds(0, _KC)], ib.at[r], si).wait()

    def compute_gidx(ib, gb, db):
      for i in range(_KC // _LANES):
        sl = pl.ds(i * _LANES, _LANES)
        gb[0, sl] = ib[0, sl] * 2 + c
        gb[1, sl] = ib[2, sl] * 2 + c
        db[sl] = ib[1, sl]

    def issue_xr(j, gb, mb, rb, sx, sr):
      pltpu.async_copy(x2rh.at[gb.at[0]], mb, sx)
      pltpu.async_copy(randh.at[pl.ds(e0 + j * _KC, _KC)], rb, sr)

    def wait_x(gb, mb, sx):
      pltpu.make_async_copy(x2rh.at[gb.at[0]], mb, sx).wait()

    def issue_emb(gb, mb, se):
      pltpu.async_copy(embs.at[gb.at[1]], mb, se, add=True)

    def wait_emb(gb, mb, se):
      pltpu.make_async_copy(embs.at[gb.at[1]], mb, se).wait()

    def wait_rand(rb, sr):
      pltpu.make_async_copy(randh.at[pl.ds(0, _KC)], rb, sr).wait()

    def relu_scatter(mb, rb, db, ss):
      @plsc.parallel_loop(0, _KC, 1, unroll=4)
      def _(r):
        for v in range(_VPR):
          sl = pl.ds(v * _LANES, _LANES)
          val = mb[r, sl]
          if v == _VPR - 1:
            val = val + rb[r, :] * cf
          mb[r, sl] = jnp.maximum(val, 0.0)

      pltpu.async_copy(mb, acc.at[db], ss, add=True)

    def wait_scatter(mb, db, ss):
      pltpu.make_async_copy(mb, acc.at[db], ss).wait()

    P0 = (idx0, gidx0, dstb0, mbuf0, rbuf0, si0, sx0, se0, sr0, ss0)
    P1 = (idx1, gidx1, dstb1, mbuf1, rbuf1, si1, sx1, se1, sr1, ss1)

    def half(j, B, Bo, wait_prev_scatter):
      (ib, gb, db, mb, rb, si, sx, se, sr, ss) = B
      (ibo, gbo, dbo, mbo, rbo, sio, sxo, seo, sro, sso) = Bo
      wait_idx(ibo, sio)
      if wait_prev_scatter is True:
        wait_scatter(mbo, dbo, sso)
      elif wait_prev_scatter is not False:
        @pl.when(wait_prev_scatter)
        def _():
          wait_scatter(mbo, dbo, sso)
      compute_gidx(ibo, gbo, dbo)
      issue_xr(j + 1, gbo, mbo, rbo, sxo, sro)
      issue_idx(j + 2, ib, si)
      wait_emb(gb, mb, se)
      wait_rand(rb, sr)
      relu_scatter(mb, rb, db, ss)
      wait_x(gbo, mbo, sxo)
      issue_emb(gbo, mbo, seo)

    pltpu.sync_copy(src3h.at[pl.ds(e0, _KC)], idx0.at[0])
    pltpu.sync_copy(dst3h.at[pl.ds(e0, _KC)], idx0.at[1])
    pltpu.sync_copy(cidxh.at[pl.ds(e0, _KC)], idx0.at[2])
    compute_gidx(idx0, gidx0, dstb0)
    issue_xr(0, gidx0, mbuf0, rbuf0, sx0, sr0)
    issue_idx(1, idx1, si1)
    wait_x(gidx0, mbuf0, sx0)
    issue_emb(gidx0, mbuf0, se0)

    def pair(g, _):
      j0 = g * 2
      half(j0, P0, P1, jnp.greater(g, 0))
      half(j0 + 1, P1, P0, True)
      return 0

    lax.fori_loop(0, (_NCHUNK - 1) // 2, pair, 0)
    wait_emb(gidx0, mbuf0, se0)
    wait_rand(rbuf0, sr0)
    relu_scatter(mbuf0, rbuf0, dstb0, ss0)
    wait_scatter(mbuf0, dstb0, ss0)
    wait_scatter(mbuf1, dstb1, ss1)
    wait_idx(idx1, si1)

    plsc.subcore_barrier()

    pltpu.sync_copy(acc.at[pl.ds(r0, _RPT)], outh.at[pl.ds(r0, _RPT), c])

  return k(x3, x2r, src3, dst3, a03, rand4, embc_r, eps16)


_NB = 10
_BR = _N // _NB


def _mlp(h, W1, b1, gamma, beta, W2, b2):
  def body(h_ref, w1_ref, b1_ref, g_ref, be_ref, w2_ref, b2_ref, o_ref,
           h1s, accs):
    p = pl.program_id(0)
    i = pl.program_id(1)

    @pl.when(p == 0)
    def _():
      h1 = jnp.dot(h_ref[:], w1_ref[:], preferred_element_type=jnp.float32)
      h1 = h1 + b1_ref[:]
      h1s[pl.ds(i * _BR, _BR), :] = h1

      @pl.when(i == 0)
      def _():
        accs[:] = jnp.zeros_like(accs)

      accs[0:1, :] = accs[0:1, :] + jnp.sum(h1, axis=0, keepdims=True)
      accs[1:2, :] = accs[1:2, :] + jnp.sum(h1 * h1, axis=0, keepdims=True)

    @pl.when(p == 1)
    def _():
      mu = accs[0:1, :] / _N
      var = accs[1:2, :] / _N - mu * mu
      inv = lax.rsqrt(var + 1e-5)
      a = (h1s[pl.ds(i * _BR, _BR), :] - mu) * (inv * g_ref[:]) + be_ref[:]
      a = jnp.maximum(a, 0.0)
      o_ref[:] = jnp.dot(a, w2_ref[:],
                         preferred_element_type=jnp.float32) + b2_ref[:]

  cst = lambda p, i: (0, 0)
  return pl.pallas_call(
      body,
      grid=(2, _NB),
      in_specs=[
          pl.BlockSpec((_BR, _H), lambda p, i: (i * (1 - p), 0)),
          pl.BlockSpec((_H, _H), cst),
          pl.BlockSpec((1, _H), cst),
          pl.BlockSpec((1, _H), cst),
          pl.BlockSpec((1, _H), cst),
          pl.BlockSpec((_H, _H), cst),
          pl.BlockSpec((1, _H), cst),
      ],
      out_specs=pl.BlockSpec((_BR, _H), lambda p, i: (i * p, 0)),
      out_shape=jax.ShapeDtypeStruct((_N, _H), jnp.float32),
      scratch_shapes=[
          pltpu.VMEM((_N, _H), jnp.float32),
          pltpu.VMEM((8, _H), jnp.float32),
      ],
  )(h, W1, b1.reshape(1, _H), gamma.reshape(1, _H), beta.reshape(1, _H),
    W2, b2.reshape(1, _H))


def kernel(x, edge_index, edge_attr, rand_edge, emb0, emb1, emb2,
           W1, b1, gamma, beta, W2, b2, eps):
  x3 = x.reshape(_N, _NC, _HALF)
  x2r = x.reshape(_N * _NC, _HALF)
  src3 = edge_index[0]
  dst3 = edge_index[1]
  cidx = jnp.dot(edge_attr.astype(jnp.float32),
                 jnp.array([12.0, 2.0, 1.0], jnp.float32)).astype(jnp.int32)
  rand4 = rand_edge
  embc = (emb0[:, None, None, :] + emb1[None, :, None, :]
          + emb2[None, None, :, :]).reshape(5 * 6 * 2, _BD)
  embc = jnp.concatenate([embc, jnp.zeros((5 * 6 * 2, _R), jnp.float32)],
                         axis=1)
  embc_r = embc.reshape(2 * 5 * 6 * 2, _HALF)
  eps16 = jnp.broadcast_to(eps, (_LANES,)).astype(jnp.float32)

  h = _sc_message_passing(x3, x2r, src3, dst3, cidx, rand4, embc_r, eps16)
  h = h.reshape(_N, _H)
  return _mlp(h, W1, b1, gamma, beta, W2, b2)
